# Initial kernel scaffold; baseline (speedup 1.0000x reference)
import jax
import jax.numpy as jnp
from jax.experimental import pallas as pl


def kernel(multimodal, adj, W, gamma, beta):
    raise NotImplementedError("write your pallas kernel here")


# speedup vs baseline: 1.1121x; 1.1121x over previous
import functools

import jax
import jax.numpy as jnp
from jax.experimental import pallas as pl
from jax.experimental.pallas import tpu as pltpu

N, M, D = 4096, 4, 128
ALPHA = 0.05
EPS = 1e-5
TM = 256


def _fused_kernel(x_full_ref, adj_ref, w_ref, gamma_ref, beta_ref, out_ref,
                  y_ref):
    i = pl.program_id(0)

    @pl.when(i == 0)
    def _build_y():
        w = w_ref[...].astype(jnp.bfloat16)
        for m in range(M):
            xm = x_full_ref[:, m, :].astype(jnp.bfloat16)
            ym = jnp.dot(xm, w, preferred_element_type=jnp.float32)
            y_ref[:, m * D:(m + 1) * D] = ym.astype(jnp.bfloat16)

    adj = adj_ref[...].astype(jnp.bfloat16)
    z = jnp.dot(adj, y_ref[...], preferred_element_type=jnp.float32)

    gamma = gamma_ref[...]
    beta = beta_ref[...]
    row0 = i * TM
    for m in range(M):
        xm = x_full_ref[pl.ds(row0, TM), m, :]
        v = xm + ALPHA * z[:, m * D:(m + 1) * D]
        mu = jnp.mean(v, axis=-1, keepdims=True)
        c = v - mu
        var = jnp.mean(c * c, axis=-1, keepdims=True)
        out_ref[:, m, :] = c * jax.lax.rsqrt(var + EPS) * gamma + beta


@functools.partial(jax.jit, static_argnames=())
def kernel(multimodal, adj, W, gamma, beta):
    n, m, d = multimodal.shape
    gamma2 = gamma.reshape(1, d)
    beta2 = beta.reshape(1, d)
    out = pl.pallas_call(
        _fused_kernel,
        grid=(n // TM,),
        in_specs=[
            pl.BlockSpec((n, m, d), lambda i: (0, 0, 0)),
            pl.BlockSpec((TM, n), lambda i: (i, 0)),
            pl.BlockSpec((d, d), lambda i: (0, 0)),
            pl.BlockSpec((1, d), lambda i: (0, 0)),
            pl.BlockSpec((1, d), lambda i: (0, 0)),
        ],
        out_specs=pl.BlockSpec((TM, m, d), lambda i: (i, 0, 0)),
        out_shape=jax.ShapeDtypeStruct((n, m, d), jnp.float32),
        scratch_shapes=[pltpu.VMEM((n, m * d), jnp.bfloat16)],
        compiler_params=pltpu.CompilerParams(
            dimension_semantics=("arbitrary",),
        ),
    )(multimodal, adj, W, gamma2, beta2)
    return out
